# baseline (speedup 1.0000x reference)
import functools

import jax
import jax.numpy as jnp
from jax import lax
from jax.experimental import pallas as pl
from jax.experimental.pallas import tpu as pltpu
from jax.experimental.pallas import tpu_sc as plsc

M = 4096
K = 8192
D = 256

BM = 2048
BK = 1024
NM = M // BM
NK = K // BK


def _dist_argmin_body(x_ref, e_ref, dist_ref, ind_ref, rmin_ref, rtile_ref):
    k = pl.program_id(1)

    x = x_ref[...]
    e = e_ref[...]
    mm = lax.dot_general(
        x, e, (((1,), (1,)), ((), ())),
        preferred_element_type=jnp.float32)
    xsq = jnp.sum(x * x, axis=1)
    esq = jnp.sum(e * e, axis=1)
    dist = (xsq[:, None] - 2.0 * mm) + esq[None, :]
    dist_ref[...] = dist

    NG = BK // 128
    dist3 = dist.reshape(BM, NG, 128)
    pmin = jnp.min(dist3, axis=1)
    pband = jnp.full((BM, 128), NG - 1, jnp.int32)
    for g in range(NG - 2, -1, -1):
        pband = jnp.where(dist3[:, g, :] == pmin, g, pband)
    pcode = k * NG + pband

    @pl.when(k == 0)
    def _init():
        rmin_ref[...] = pmin
        rtile_ref[...] = pcode

    @pl.when(k != 0)
    def _update():
        prev = rmin_ref[...]
        better = pmin < prev
        rmin_ref[...] = jnp.where(better, pmin, prev)
        rtile_ref[...] = jnp.where(better, pcode, rtile_ref[...])

    @pl.when(k == NK - 1)
    def _final():
        rmin = rmin_ref[...]
        rowmin = jnp.min(rmin, axis=1)
        lanes = lax.broadcasted_iota(jnp.int32, (BM, 128), 1)
        gidx = rtile_ref[...] * 128 + lanes
        ind_ref[...] = jnp.min(
            jnp.where(rmin == rowmin[:, None], gidx, jnp.int32(K)), axis=1)


def _dist_argmin(flat_x, embed):
    return pl.pallas_call(
        _dist_argmin_body,
        grid=(NM, NK),
        in_specs=[
            pl.BlockSpec((BM, D), lambda m, k: (m, 0)),
            pl.BlockSpec((BK, D), lambda m, k: (k, 0)),
        ],
        out_specs=[
            pl.BlockSpec((BM, BK), lambda m, k: (m, k)),
            pl.BlockSpec((BM,), lambda m, k: (m,)),
        ],
        out_shape=[
            jax.ShapeDtypeStruct((M, K), jnp.float32),
            jax.ShapeDtypeStruct((M,), jnp.int32),
        ],
        scratch_shapes=[
            pltpu.VMEM((BM, 128), jnp.float32),
            pltpu.VMEM((BM, 128), jnp.int32),
        ],
    )(flat_x, embed)


_SC_INFO = plsc.get_sparse_core_info()
_NW = _SC_INFO.num_cores * _SC_INFO.num_subcores
_B_PER_W = M // _NW

_sc_mesh = plsc.VectorSubcoreMesh(core_axis_name="c", subcore_axis_name="s")


@functools.partial(
    pl.kernel,
    mesh=_sc_mesh,
    out_type=jax.ShapeDtypeStruct((M, D), jnp.float32),
    scratch_types=[
        pltpu.VMEM((_B_PER_W,), jnp.int32),
        pltpu.VMEM((_B_PER_W, D), jnp.float32),
        pltpu.SemaphoreType.DMA,
    ],
)
def _sc_gather(table_hbm, idx_hbm, out_hbm, idx_v, rows_v, sem):
    wid = lax.axis_index("s") * _SC_INFO.num_cores + lax.axis_index("c")
    base = wid * _B_PER_W
    pltpu.sync_copy(idx_hbm.at[pl.ds(base, _B_PER_W)], idx_v)
    pltpu.async_copy(table_hbm.at[idx_v], rows_v, sem).wait()
    pltpu.sync_copy(rows_v, out_hbm.at[pl.ds(base, _B_PER_W)])


def kernel(x, embed):
    x = x.astype(jnp.float32)
    shape = x.shape
    flat_x = x.reshape(M, D)
    dist_sq, ind = _dist_argmin(flat_x, embed)
    quantize = _sc_gather(embed, ind)
    num_expired = jnp.zeros((), dtype=jnp.int32)
    return (
        quantize.reshape(shape),
        ind.reshape(shape[:-1]),
        num_expired,
        dist_sq.reshape(*shape[:-1], K),
    )


# speedup vs baseline: 2.6840x; 2.6840x over previous
import functools

import jax
import jax.numpy as jnp
from jax import lax
from jax.experimental import pallas as pl
from jax.experimental.pallas import tpu as pltpu
from jax.experimental.pallas import tpu_sc as plsc

M = 4096
K = 8192
D = 256

BM = 2048
BK = 1024
NM = M // BM
NK = K // BK


def _dist_argmin_body(x_ref, e_ref, dist_ref, ind_ref, rmin_ref, rtile_ref):
    k = pl.program_id(1)

    x = x_ref[...]
    e = e_ref[...]
    mm = lax.dot_general(
        x, e, (((1,), (1,)), ((), ())),
        preferred_element_type=jnp.float32)
    xsq = jnp.sum(x * x, axis=1)
    esq = jnp.sum(e * e, axis=1)
    dist = (xsq[:, None] - 2.0 * mm) + esq[None, :]
    dist_ref[...] = dist

    NG = BK // 128
    parts = [dist[:, g * 128:(g + 1) * 128] for g in range(NG)]
    pmin = parts[0]
    for g in range(1, NG):
        pmin = jnp.minimum(pmin, parts[g])
    pband = jnp.full((BM, 128), NG - 1, jnp.int32)
    for g in range(NG - 2, -1, -1):
        pband = jnp.where(parts[g] == pmin, g, pband)
    pcode = k * NG + pband

    @pl.when(k == 0)
    def _init():
        rmin_ref[...] = pmin
        rtile_ref[...] = pcode

    @pl.when(k != 0)
    def _update():
        prev = rmin_ref[...]
        better = pmin < prev
        rmin_ref[...] = jnp.where(better, pmin, prev)
        rtile_ref[...] = jnp.where(better, pcode, rtile_ref[...])

    @pl.when(k == NK - 1)
    def _final():
        rmin = rmin_ref[...]
        rowmin = jnp.min(rmin, axis=1)
        lanes = lax.broadcasted_iota(jnp.int32, (BM, 128), 1)
        gidx = rtile_ref[...] * 128 + lanes
        ind_ref[...] = jnp.min(
            jnp.where(rmin == rowmin[:, None], gidx, jnp.int32(K)), axis=1)


def _dist_argmin(flat_x, embed):
    return pl.pallas_call(
        _dist_argmin_body,
        grid=(NM, NK),
        in_specs=[
            pl.BlockSpec((BM, D), lambda m, k: (m, 0)),
            pl.BlockSpec((BK, D), lambda m, k: (k, 0)),
        ],
        out_specs=[
            pl.BlockSpec((BM, BK), lambda m, k: (m, k)),
            pl.BlockSpec((BM,), lambda m, k: (m,)),
        ],
        out_shape=[
            jax.ShapeDtypeStruct((M, K), jnp.float32),
            jax.ShapeDtypeStruct((M,), jnp.int32),
        ],
        scratch_shapes=[
            pltpu.VMEM((BM, 128), jnp.float32),
            pltpu.VMEM((BM, 128), jnp.int32),
        ],
    )(flat_x, embed)


_SC_INFO = plsc.get_sparse_core_info()
_NW = _SC_INFO.num_cores * _SC_INFO.num_subcores
_B_PER_W = M // _NW

_sc_mesh = plsc.VectorSubcoreMesh(core_axis_name="c", subcore_axis_name="s")


@functools.partial(
    pl.kernel,
    mesh=_sc_mesh,
    out_type=jax.ShapeDtypeStruct((M, D), jnp.float32),
    scratch_types=[
        pltpu.VMEM((_B_PER_W,), jnp.int32),
        pltpu.VMEM((_B_PER_W, D), jnp.float32),
        pltpu.SemaphoreType.DMA,
    ],
)
def _sc_gather(table_hbm, idx_hbm, out_hbm, idx_v, rows_v, sem):
    wid = lax.axis_index("s") * _SC_INFO.num_cores + lax.axis_index("c")
    base = wid * _B_PER_W
    pltpu.sync_copy(idx_hbm.at[pl.ds(base, _B_PER_W)], idx_v)
    pltpu.async_copy(table_hbm.at[idx_v], rows_v, sem).wait()
    pltpu.sync_copy(rows_v, out_hbm.at[pl.ds(base, _B_PER_W)])


def kernel(x, embed):
    x = x.astype(jnp.float32)
    shape = x.shape
    flat_x = x.reshape(M, D)
    dist_sq, ind = _dist_argmin(flat_x, embed)
    quantize = _sc_gather(embed, ind)
    num_expired = jnp.zeros((), dtype=jnp.int32)
    return (
        quantize.reshape(shape),
        ind.reshape(shape[:-1]),
        num_expired,
        dist_sq.reshape(*shape[:-1], K),
    )
